# Initial kernel scaffold; baseline (speedup 1.0000x reference)
import jax
import jax.numpy as jnp
from jax.experimental import pallas as pl


def kernel(velocity_length, velocity_theta, a_len, a_wid, a_type, position, heading, l_embs, visible_mask, a_batch, W1a, b1a, W2a, b2a, We1, be1, We2, be2, Wq, Wk, Wv, Wke, Wve, Wo, Wt1, bt1, Wt2, bt2):
    raise NotImplementedError("write your pallas kernel here")


# speedup vs baseline: 175.3094x; 175.3094x over previous
import functools

import jax
import jax.numpy as jnp
from jax.experimental import pallas as pl

_NH = 8


def _fold_kernel(We2T, Wke_T, Wve_T, be2T, out_Wke2T, out_Wve2T,
                 out_kbiasT, out_vbiasT):
    out_Wke2T[...] = jnp.dot(Wke_T[...], We2T[...], preferred_element_type=jnp.float32)
    out_Wve2T[...] = jnp.dot(Wve_T[...], We2T[...], preferred_element_type=jnp.float32)
    out_kbiasT[...] = jnp.dot(Wke_T[...], be2T[...], preferred_element_type=jnp.float32)
    out_vbiasT[...] = jnp.dot(Wve_T[...], be2T[...], preferred_element_type=jnp.float32)


def _phase1_kernel(a_n, a_T, gin_n, gin_T, W1a, b1a, W1aT, b1aT, W2a, b2a,
                   W2aT, b2aT, Wq, WkT, WvT, kbiasT, vbiasT,
                   x_out, q_out, kbT_out, vbT_out, geon_out, geoT_out):
    an = a_n[0]
    aT = a_T[0]
    nn = an.shape[0]
    d = W2a.shape[0]
    acc = jnp.broadcast_to(b1a[...], (nn, d))
    for f in range(5):
        acc = acc + an[:, f:f + 1] * W1a[f:f + 1, :]
    x = jnp.dot(jnp.maximum(acc, 0.0), W2a[...],
                preferred_element_type=jnp.float32) + b2a[...]
    x_out[0] = x
    q_out[0] = jnp.dot(x, Wq[...], preferred_element_type=jnp.float32)
    accT = jnp.broadcast_to(b1aT[...], (d, nn))
    for f in range(5):
        accT = accT + W1aT[:, f:f + 1] * aT[f:f + 1, :]
    xT = jnp.dot(W2aT[...], jnp.maximum(accT, 0.0),
                 preferred_element_type=jnp.float32) + b2aT[...]
    kbT_out[0] = jnp.dot(WkT[...], xT, preferred_element_type=jnp.float32) + kbiasT[...]
    vbT_out[0] = jnp.dot(WvT[...], xT, preferred_element_type=jnp.float32) + vbiasT[...]
    geon_out[0, :, 0:2] = gin_n[0, :, 0:2]
    geon_out[0, :, 2:3] = jnp.cos(gin_n[0, :, 2:3])
    geon_out[0, :, 3:4] = jnp.sin(gin_n[0, :, 2:3])
    geon_out[0, :, 4:8] = gin_n[0, :, 3:7]
    geoT_out[0, 0:2, :] = gin_T[0, 0:2, :]
    geoT_out[0, 2:3, :] = jnp.cos(gin_T[0, 2:3, :])
    geoT_out[0, 3:4, :] = jnp.sin(gin_T[0, 2:3, :])
    geoT_out[0, 4:8, :] = gin_T[0, 3:7, :]


def _phase2_kernel(q_ref, kbT_ref, vbT_ref, gd_ref, gs_ref, We1T, be1T,
                   Wke2T, Wve2T, agg_ref, *, at, nsrc, nh):
    d = Wke2T.shape[0]
    hd = d // nh
    qd = q_ref[0]
    kbT = kbT_ref[0]
    vbT = vbT_ref[0]
    gd = gd_ref[0]
    gs = gs_ref[0]

    pxs = gs[0:1, :]
    pys = gs[1:2, :]
    cs = gs[2:3, :]
    ss = gs[3:4, :]
    viss = gs[4:5, :]
    bs = gs[5:6, :]
    bid = jax.lax.broadcasted_iota(jnp.int32, (1, nsrc), 1)
    hmask = (jax.lax.broadcasted_iota(jnp.int32, (nh, d), 1) // hd
             == jax.lax.broadcasted_iota(jnp.int32, (nh, d), 0)).astype(jnp.float32)
    ai = pl.program_id(1)
    scale = 1.0 / (hd ** 0.5)

    for a in range(at):
        pxd = gd[a:a + 1, 0:1]
        pyd = gd[a:a + 1, 1:2]
        cd = gd[a:a + 1, 2:3]
        sd = gd[a:a + 1, 3:4]
        visd = gd[a:a + 1, 4:5]
        bd = gd[a:a + 1, 5:6]
        relx = pxs - pxd
        rely = pys - pyd
        lx = cd * relx + sd * rely
        ly = -sd * relx + cd * rely
        dist = jnp.sqrt(lx * lx + ly * ly + 1e-12)
        ang = jnp.arctan2(ly, lx + 1e-12)
        sin_dhd = ss * cd - cs * sd
        cos_dhd = cs * cd + ss * sd
        dhd = jnp.arctan2(sin_dhd, cos_dhd)

        H = jnp.maximum(We1T[:, 0:1] * dist + We1T[:, 1:2] * ang
                        + We1T[:, 2:3] * dhd + be1T[...], 0.0)
        kfT = jnp.dot(Wke2T[...], H, preferred_element_type=jnp.float32) + kbT
        vfT = jnp.dot(Wve2T[...], H, preferred_element_type=jnp.float32) + vbT

        qexp = jnp.broadcast_to(qd[a:a + 1, :], (nh, d)) * hmask
        logits = jnp.dot(qexp, kfT, preferred_element_type=jnp.float32) * scale

        aid = ai * at + a
        mask = ((bid != aid) & (bs == bd) & (visd > 0.5)
                & (viss > 0.5))
        neg = jnp.float32(-jnp.inf)
        ml = jnp.where(mask, logits, neg)
        mx = jnp.max(ml, axis=1, keepdims=True)
        e = jnp.where(mask, jnp.exp(logits - mx), 0.0)
        den = jnp.sum(e, axis=1, keepdims=True)
        alpha = e / (den + 1e-16)

        m2 = jax.lax.dot_general(alpha, vfT, (((1,), (1,)), ((), ())),
                                 preferred_element_type=jnp.float32)
        agg_ref[0, a:a + 1, :] = jnp.sum(m2 * hmask, axis=0, keepdims=True)


def _phase3_kernel(x_ref, agg_ref, Wo, Wt1, bt1, Wt2, bt2, traj_ref):
    x = x_ref[0]
    out = x + jnp.dot(agg_ref[0], Wo[...], preferred_element_type=jnp.float32)
    h = jnp.maximum(jnp.dot(out, Wt1[...], preferred_element_type=jnp.float32)
                    + bt1[...], 0.0)
    traj_ref[0] = (jnp.dot(h, Wt2[...], preferred_element_type=jnp.float32)
                   + bt2[...])


def kernel(velocity_length, velocity_theta, a_len, a_wid, a_type, position,
           heading, l_embs, visible_mask, a_batch, W1a, b1a, W2a, b2a, We1,
           be1, We2, be2, Wq, Wk, Wv, Wke, Wve, Wo, Wt1, bt1, Wt2, bt2):
    n, t = velocity_length.shape
    d = W2a.shape[1]
    nh = _NH
    nf2 = Wt2.shape[1]
    f32 = jnp.float32

    at = 8
    for cand in (8, 10, 16, 20, 25, 5, 4, 2, 1):
        if n % cand == 0:
            at = cand
            break

    Wke2T, Wve2T, kbiasT, vbiasT = pl.pallas_call(
        _fold_kernel,
        out_shape=[
            jax.ShapeDtypeStruct((d, d), f32),
            jax.ShapeDtypeStruct((d, d), f32),
            jax.ShapeDtypeStruct((d, 1), f32),
            jax.ShapeDtypeStruct((d, 1), f32),
        ],
    )(We2.T, Wke.T, Wve.T, be2.reshape(d, 1))

    ones_t = jnp.ones((n, t), f32)
    feats = jnp.stack([velocity_length, velocity_theta, a_len[:, None] * ones_t,
                       a_wid[:, None] * ones_t, a_type[:, None] * ones_t],
                      axis=0)
    a_in_T = feats.transpose(2, 0, 1)
    a_in_n = feats.transpose(2, 1, 0)
    graw = jnp.stack([position[:, :, 0], position[:, :, 1], heading,
                      visible_mask.astype(f32),
                      jnp.broadcast_to(a_batch.astype(f32)[:, None], (n, t)),
                      jnp.zeros((n, t), f32), jnp.zeros((n, t), f32)],
                     axis=0)
    gin_T = graw.transpose(2, 0, 1)
    gin_n = graw.transpose(2, 1, 0)

    node_shape = jax.ShapeDtypeStruct((t, n, d), f32)
    nodeT_shape = jax.ShapeDtypeStruct((t, d, n), f32)
    wspec = pl.BlockSpec((d, d), lambda i: (0, 0))
    rspec = pl.BlockSpec((1, d), lambda i: (0, 0))
    cspec = pl.BlockSpec((d, 1), lambda i: (0, 0))
    x_t, q_t, kbT_t, vbT_t, geon_t, geoT_t = pl.pallas_call(
        _phase1_kernel,
        grid=(t,),
        in_specs=[
            pl.BlockSpec((1, n, 5), lambda i: (i, 0, 0)),
            pl.BlockSpec((1, 5, n), lambda i: (i, 0, 0)),
            pl.BlockSpec((1, n, 7), lambda i: (i, 0, 0)),
            pl.BlockSpec((1, 7, n), lambda i: (i, 0, 0)),
            pl.BlockSpec((5, d), lambda i: (0, 0)),
            rspec,
            pl.BlockSpec((d, 5), lambda i: (0, 0)),
            cspec,
            wspec,
            rspec,
            wspec,
            cspec,
            wspec,
            wspec,
            wspec,
            cspec,
            cspec,
        ],
        out_specs=[
            pl.BlockSpec((1, n, d), lambda i: (i, 0, 0)),
            pl.BlockSpec((1, n, d), lambda i: (i, 0, 0)),
            pl.BlockSpec((1, d, n), lambda i: (i, 0, 0)),
            pl.BlockSpec((1, d, n), lambda i: (i, 0, 0)),
            pl.BlockSpec((1, n, 8), lambda i: (i, 0, 0)),
            pl.BlockSpec((1, 8, n), lambda i: (i, 0, 0)),
        ],
        out_shape=[node_shape, node_shape, nodeT_shape, nodeT_shape,
                   jax.ShapeDtypeStruct((t, n, 8), f32),
                   jax.ShapeDtypeStruct((t, 8, n), f32)],
    )(a_in_n, a_in_T, gin_n, gin_T, W1a, b1a.reshape(1, d), W1a.T,
      b1a.reshape(d, 1), W2a, b2a.reshape(1, d), W2a.T, b2a.reshape(d, 1),
      Wq, Wk.T, Wv.T, kbiasT, vbiasT)

    agg49 = pl.pallas_call(
        functools.partial(_phase2_kernel, at=at, nsrc=n, nh=nh),
        grid=(t - 1, n // at),
        in_specs=[
            pl.BlockSpec((1, at, d), lambda ti, ai: (ti + 1, ai, 0)),
            pl.BlockSpec((1, d, n), lambda ti, ai: (ti, 0, 0)),
            pl.BlockSpec((1, d, n), lambda ti, ai: (ti, 0, 0)),
            pl.BlockSpec((1, at, 8), lambda ti, ai: (ti + 1, ai, 0)),
            pl.BlockSpec((1, 8, n), lambda ti, ai: (ti, 0, 0)),
            pl.BlockSpec((d, 3), lambda ti, ai: (0, 0)),
            pl.BlockSpec((d, 1), lambda ti, ai: (0, 0)),
            pl.BlockSpec((d, d), lambda ti, ai: (0, 0)),
            pl.BlockSpec((d, d), lambda ti, ai: (0, 0)),
        ],
        out_specs=pl.BlockSpec((1, at, d), lambda ti, ai: (ti, ai, 0)),
        out_shape=jax.ShapeDtypeStruct((t - 1, n, d), f32),
    )(q_t, kbT_t, vbT_t, geon_t, geoT_t, We1.T, be1.reshape(d, 1),
      Wke2T, Wve2T)

    agg_full = jnp.concatenate([jnp.zeros((1, n, d), f32), agg49], axis=0)

    traj_t = pl.pallas_call(
        _phase3_kernel,
        grid=(t,),
        in_specs=[
            pl.BlockSpec((1, n, d), lambda i: (i, 0, 0)),
            pl.BlockSpec((1, n, d), lambda i: (i, 0, 0)),
            wspec,
            wspec,
            rspec,
            pl.BlockSpec((d, nf2), lambda i: (0, 0)),
            pl.BlockSpec((1, nf2), lambda i: (0, 0)),
        ],
        out_specs=pl.BlockSpec((1, n, nf2), lambda i: (i, 0, 0)),
        out_shape=jax.ShapeDtypeStruct((t, n, nf2), f32),
    )(x_t, agg_full, Wo, Wt1, bt1.reshape(1, d), Wt2, bt2.reshape(1, nf2))

    return traj_t.transpose(1, 0, 2)
